# baseline (speedup 1.0000x reference)
import jax
import jax.numpy as jnp
from jax.experimental import pallas as pl
from jax.experimental.pallas import tpu as pltpu

DIM = 1024
SCALE = DIM ** (-0.5)

_ROWS = 8192
_CHUNK = 256
_NBUF = 8
_NCHUNK = _ROWS // _CHUNK
_LAG = 4


def _tc_body(w_hbm, o_hbm, buf, in_sem, out_sem):
    def start_in(i):
        b = i % _NBUF
        return pltpu.make_async_copy(
            w_hbm.at[pl.ds(i * _CHUNK, _CHUNK)],
            buf.at[b],
            in_sem.at[b],
        )

    def start_out(i):
        b = i % _NBUF
        return pltpu.make_async_copy(
            buf.at[b],
            o_hbm.at[pl.ds(i * _CHUNK, _CHUNK)],
            out_sem.at[b],
        )

    unwaited = set()
    for i in range(_NBUF):
        start_in(i).start()

    for i in range(_NCHUNK):
        b = i % _NBUF
        j = i - _LAG
        if j >= 0 and j + _NBUF < _NCHUNK:
            start_out(j).wait()
            unwaited.discard(j)
            start_in(j + _NBUF).start()

        start_in(i).wait()
        buf[b] = buf[b] * SCALE
        start_out(i).start()
        unwaited.add(i)

    for i in sorted(unwaited):
        start_out(i).wait()


def kernel(x, W):
    n = x.shape[1]
    return pl.pallas_call(
        _tc_body,
        in_specs=[pl.BlockSpec(memory_space=pl.ANY)],
        out_specs=pl.BlockSpec(memory_space=pl.ANY),
        out_shape=jax.ShapeDtypeStruct((n, DIM), W.dtype),
        scratch_shapes=[
            pltpu.VMEM((_NBUF, _CHUNK, DIM), jnp.float32),
            pltpu.SemaphoreType.DMA((_NBUF,)),
            pltpu.SemaphoreType.DMA((_NBUF,)),
        ],
    )(W[:n])


# speedup vs baseline: 1.0222x; 1.0222x over previous
import jax
import jax.numpy as jnp
from jax.experimental import pallas as pl
from jax.experimental.pallas import tpu as pltpu

DIM = 1024
SCALE = DIM ** (-0.5)

_ROWS = 8192
_CHUNK = 1024
_NBUF = 8
_NCHUNK = _ROWS // _CHUNK
_LAG = 4


def _tc_body(w_hbm, o_hbm, buf, in_sem, out_sem):
    def start_in(i):
        b = i % _NBUF
        return pltpu.make_async_copy(
            w_hbm.at[pl.ds(i * _CHUNK, _CHUNK)],
            buf.at[b],
            in_sem.at[b],
        )

    def start_out(i):
        b = i % _NBUF
        return pltpu.make_async_copy(
            buf.at[b],
            o_hbm.at[pl.ds(i * _CHUNK, _CHUNK)],
            out_sem.at[b],
        )

    unwaited = set()
    for i in range(_NBUF):
        start_in(i).start()

    for i in range(_NCHUNK):
        b = i % _NBUF
        j = i - _LAG
        if j >= 0 and j + _NBUF < _NCHUNK:
            start_out(j).wait()
            unwaited.discard(j)
            start_in(j + _NBUF).start()

        start_in(i).wait()
        buf[b] = buf[b] * SCALE
        start_out(i).start()
        unwaited.add(i)

    for i in sorted(unwaited):
        start_out(i).wait()


def kernel(x, W):
    n = x.shape[1]
    return pl.pallas_call(
        _tc_body,
        in_specs=[pl.BlockSpec(memory_space=pl.ANY)],
        out_specs=pl.BlockSpec(memory_space=pl.ANY),
        out_shape=jax.ShapeDtypeStruct((n, DIM), W.dtype),
        scratch_shapes=[
            pltpu.VMEM((_NBUF, _CHUNK, DIM), jnp.float32),
            pltpu.SemaphoreType.DMA((_NBUF,)),
            pltpu.SemaphoreType.DMA((_NBUF,)),
        ],
    )(W[:n])


# speedup vs baseline: 1.0269x; 1.0047x over previous
import jax
import jax.numpy as jnp
from jax.experimental import pallas as pl

DIM = 1024
SCALE = DIM ** (-0.5)


def _scale_kernel(w_ref, o_ref):
    o_ref[...] = w_ref[...] * SCALE


def kernel(x, W):
    n = x.shape[1]
    rows_per_block = 2048
    grid = (n // rows_per_block,)
    return pl.pallas_call(
        _scale_kernel,
        grid=grid,
        in_specs=[pl.BlockSpec((rows_per_block, DIM), lambda i: (i, 0))],
        out_specs=pl.BlockSpec((rows_per_block, DIM), lambda i: (i, 0)),
        out_shape=jax.ShapeDtypeStruct((n, DIM), W.dtype),
    )(W[:n])
